# baseline (speedup 1.0000x reference)
import jax
import jax.numpy as jnp
from jax.experimental import pallas as pl
from jax.experimental.pallas import tpu as pltpu

_KEEP_RATIO = 0.7
_ZSCORE_EPS = 1e-3
_EPS = 1e-6


def _stats_gate_kernel(x_ref, rm_ref, fm_ref, rs_ref, fs_ref, gate_ref,
                       ia_ref, *, n_sub, c, hw, kth):
    step = pl.program_id(0)
    nsteps = pl.num_programs(0)

    @pl.when(step == 0)
    def _init():
        ia_ref[...] = jnp.zeros_like(ia_ref)

    xb = x_ref[...]
    m = jnp.sum(xb, axis=1, keepdims=True) * (1.0 / hw)
    rm = rm_ref[...]
    fm = fm_ref[...]
    rs = rs_ref[...]
    fs = fs_ref[...]
    disc = jnp.abs(fm - rm)
    acc = jnp.zeros((c, 1), jnp.float32)
    for i in range(n_sub):
        mi = m[i * c:(i + 1) * c, :]
        zr = jnp.abs((mi - rm) / (rs + _ZSCORE_EPS))
        zf = jnp.abs((mi - fm) / (fs + _ZSCORE_EPS))
        acc = acc + disc / (jnp.minimum(zr, zf) + _EPS)
    ia_ref[...] += acc

    @pl.when(step == nsteps - 1)
    def _gate():
        total_b = nsteps * n_sub
        v = ia_ref[...] * (1.0 / total_b)
        ones = jnp.ones((c, 1), jnp.float32)
        m1 = jax.lax.dot_general(ones, v, (((1,), (1,)), ((), ())),
                                 preferred_element_type=jnp.float32)
        m2 = jnp.broadcast_to(v, (c, c))
        cnt = jnp.sum((m1 <= m2).astype(jnp.float32), axis=1, keepdims=True)
        gate_ref[...] = (cnt >= float(kth + 1)).astype(jnp.float32)


def _mul_kernel(x_ref, gate_ref, out_ref, *, n_sub, c):
    g = gate_ref[...]
    for i in range(n_sub):
        out_ref[i * c:(i + 1) * c, :] = x_ref[i * c:(i + 1) * c, :] * g


def kernel(x, real_mean, fake_mean, real_std, fake_std):
    B, C, H, W = x.shape
    HW = H * W
    kth = max(0, min(int((1.0 - _KEEP_RATIO) * C), C - 1))

    x2 = x.reshape(B * C, HW)
    rm = real_mean.reshape(C, 1)
    fm = fake_mean.reshape(C, 1)
    rs = real_std.reshape(C, 1)
    fs = fake_std.reshape(C, 1)

    n_sub = 4
    grid = (B // n_sub,)
    rows = n_sub * C

    import functools
    stats_fn = functools.partial(_stats_gate_kernel, n_sub=n_sub, c=C, hw=HW,
                                 kth=kth)
    gate = pl.pallas_call(
        stats_fn,
        grid=grid,
        in_specs=[
            pl.BlockSpec((rows, HW), lambda i: (i, 0)),
            pl.BlockSpec((C, 1), lambda i: (0, 0)),
            pl.BlockSpec((C, 1), lambda i: (0, 0)),
            pl.BlockSpec((C, 1), lambda i: (0, 0)),
            pl.BlockSpec((C, 1), lambda i: (0, 0)),
        ],
        out_shape=jax.ShapeDtypeStruct((C, 1), jnp.float32),
        out_specs=pl.BlockSpec((C, 1), lambda i: (0, 0)),
        scratch_shapes=[pltpu.VMEM((C, 1), jnp.float32)],
    )(x2, rm, fm, rs, fs)

    mul_fn = functools.partial(_mul_kernel, n_sub=n_sub, c=C)
    out2 = pl.pallas_call(
        mul_fn,
        grid=grid,
        in_specs=[
            pl.BlockSpec((rows, HW), lambda i: (i, 0)),
            pl.BlockSpec((C, 1), lambda i: (0, 0)),
        ],
        out_shape=jax.ShapeDtypeStruct((B * C, HW), jnp.float32),
        out_specs=pl.BlockSpec((rows, HW), lambda i: (i, 0)),
    )(x2, gate)

    return out2.reshape(B, C, H, W), gate.reshape(C)
